# baseline (speedup 1.0000x reference)
import functools

import jax
import jax.numpy as jnp
from jax import lax
from jax.experimental import pallas as pl
from jax.experimental.pallas import tpu as pltpu
from jax.experimental.pallas import tpu_sc as plsc

_VOCAB = 32002
_HIDDEN = 2048
_BATCH = 4
_SEQ = 4096
_NTOK = _BATCH * _SEQ
_NW = 32
_PER_W = _NTOK // _NW
_CHUNK = 8
_NCHUNK = _PER_W // _CHUNK
_NBUF = 3

_mesh = plsc.VectorSubcoreMesh(core_axis_name="c", subcore_axis_name="s")


@functools.partial(
    pl.kernel,
    out_type=jax.ShapeDtypeStruct((_NTOK, _HIDDEN), jnp.float32),
    mesh=_mesh,
    scratch_types=(
        [pltpu.VMEM((_NCHUNK, _CHUNK), jnp.int32)]
        + [pltpu.VMEM((_CHUNK, _HIDDEN), jnp.float32)] * _NBUF
        + [pltpu.VMEM_SHARED((16, _NBUF, _CHUNK, _HIDDEN), jnp.float32)]
        + [pltpu.SemaphoreType.DMA] * (3 * _NBUF)
    ),
)
def _embed_lookup(table_hbm, idx_hbm, out_hbm, idx_v, *scratch):
    sid = lax.axis_index("s")
    wid = sid * 2 + lax.axis_index("c")
    base = wid * _PER_W
    pltpu.sync_copy(idx_hbm.at[wid], idx_v)

    bufs = scratch[:_NBUF]
    shared = scratch[_NBUF]
    gsems = scratch[_NBUF + 1:2 * _NBUF + 1]
    xsems = scratch[2 * _NBUF + 1:3 * _NBUF + 1]
    osems = scratch[3 * _NBUF + 1:]

    def gather_start(j, b):
        pltpu.async_copy(table_hbm.at[idx_v.at[j]], bufs[b], gsems[b])

    def gather_wait(b):
        pltpu.make_async_copy(
            table_hbm.at[idx_v.at[0]], bufs[b], gsems[b]).wait()

    def stage_start(b):
        pltpu.async_copy(bufs[b], shared.at[sid, b], xsems[b])

    def stage_wait(b):
        pltpu.make_async_copy(bufs[b], shared.at[sid, b], xsems[b]).wait()

    def out_start(j, b):
        pltpu.async_copy(
            shared.at[sid, b],
            out_hbm.at[pl.ds(base + j * _CHUNK, _CHUNK)], osems[b])

    def out_wait(b):
        pltpu.make_async_copy(
            shared.at[sid, b],
            out_hbm.at[pl.ds(base, _CHUNK)], osems[b]).wait()

    def slot_body(j, p, skip_out_wait=False, skip_retire=False,
                  prefetch=True):
        b = p
        bp = (p - 1) % _NBUF
        if not skip_out_wait:
            out_wait(b)
        gather_wait(b)
        stage_start(b)
        if not skip_retire:
            stage_wait(bp)
            out_start(j - 1, bp)
            if prefetch:
                gather_start(j + 2, bp)

    for b in range(_NBUF):
        gather_start(b, b)
    slot_body(0, 0, skip_out_wait=True, skip_retire=True)
    slot_body(1, 1, skip_out_wait=True)
    slot_body(2, 2, skip_out_wait=True)

    def step(k, carry):
        for p in range(_NBUF):
            slot_body(_NBUF * k + p, p)
        return carry

    _KMAX = 19
    lax.fori_loop(1, _KMAX + 1, step, 0)

    slot_body(60, 0)
    slot_body(61, 1)
    slot_body(62, 2, prefetch=False)
    slot_body(63, 0, prefetch=False)
    stage_wait(0)
    out_start(_NCHUNK - 1, 0)
    out_wait(1)
    out_wait(2)
    out_wait(0)


def kernel(input_ids, embedding_table):
    ids = input_ids.reshape(_NW, _NCHUNK, _CHUNK)
    out = _embed_lookup(embedding_table, ids)
    return out.reshape(_BATCH, _SEQ, _HIDDEN)
